# baseline (speedup 1.0000x reference)
import functools

import jax
import jax.numpy as jnp
from jax import lax
from jax.experimental import pallas as pl
from jax.experimental.pallas import tpu as pltpu
from jax.experimental.pallas import tpu_sc as plsc

CHUNK = 128
DP = 128
KA = 80


def _make_rowize(D, V, BN):
    def tr(t_ref, o_ref):
        o_ref[:, : D] = jnp.swapaxes(t_ref[...], 0, 1)

    return pl.pallas_call(
        tr,
        grid=(pl.cdiv(V, BN),),
        in_specs=[pl.BlockSpec((D, BN), lambda j: (0, j))],
        out_specs=pl.BlockSpec((BN, DP), lambda j: (j, 0)),
        out_shape=jax.ShapeDtypeStruct((V, DP), jnp.float32),
    )


@functools.lru_cache(maxsize=None)
def _make_pool(V, D, B, L):
    info = plsc.get_sparse_core_info()
    NC, NS, LN = info.num_cores, info.num_subcores, info.num_lanes
    NW = NC * NS
    assert B % NW == 0 and D % LN == 0 and KA % LN == 0
    b_per_w = B // NW
    n_idx = b_per_w * L
    assert n_idx % CHUNK == 0
    n_ch = n_idx // CHUNK
    mesh = plsc.VectorSubcoreMesh(core_axis_name="c", subcore_axis_name="s")

    @functools.partial(
        pl.kernel,
        mesh=mesh,
        out_type=jax.ShapeDtypeStruct((B, KA), jnp.float32),
        scratch_types=[
            pltpu.VMEM((n_idx,), jnp.int32),
            pltpu.VMEM((n_idx, DP), jnp.float32),
            pltpu.VMEM((b_per_w, KA), jnp.float32),
            pltpu.SemaphoreType.DMA,
        ],
    )
    def pool(idx_hbm, tbl_hbm, out_hbm, idx_v, rows_v, acc_v, sem):
        wid = lax.axis_index("s") * NC + lax.axis_index("c")
        pltpu.sync_copy(idx_hbm.at[pl.ds(wid * n_idx, n_idx)], idx_v)
        copies = []
        for j in range(n_ch):
            copies.append(
                pltpu.async_copy(
                    tbl_hbm.at[idx_v.at[pl.ds(j * CHUNK, CHUNK)]],
                    rows_v.at[pl.ds(j * CHUNK, CHUNK)],
                    sem,
                )
            )
        onez = jnp.where(
            lax.iota(jnp.int32, LN) == 0, 1.0, 0.0
        ).astype(jnp.float32)

        def body(k, carry):
            r0 = k * L
            for d in range(D // LN):
                a = rows_v[r0, pl.ds(d * LN, LN)]
                for l in range(1, L):
                    a = a + rows_v[r0 + l, pl.ds(d * LN, LN)]
                acc_v[k, pl.ds(d * LN, LN)] = a * (1.0 / L)
            for d in range(D // LN, KA // LN):
                acc_v[k, pl.ds(d * LN, LN)] = onez if d == D // LN else (
                    jnp.zeros((LN,), jnp.float32)
                )
            return carry

        k_done = 0
        for j in range(n_ch):
            copies[j].wait()
            k_ready = min((CHUNK * (j + 1)) // L, b_per_w)
            if k_ready > k_done:
                lax.fori_loop(k_done, k_ready, body, 0)
                k_done = k_ready
        pltpu.sync_copy(acc_v, out_hbm.at[pl.ds(wid * b_per_w, b_per_w)])

    return pool, NW


def _make_matmul(B, D, V, BN):
    def mm(x_ref, w_ref, b_ref, o_ref):
        lhs = jnp.concatenate(
            [w_ref[...], b_ref[...], jnp.zeros((KA - D - 1, BN), jnp.float32)],
            axis=0,
        )
        o_ref[...] = lax.dot_general(
            lhs,
            x_ref[...],
            (((0,), (1,)), ((), ())),
            preferred_element_type=jnp.float32,
        )

    return pl.pallas_call(
        mm,
        grid=(pl.cdiv(V, BN),),
        in_specs=[
            pl.BlockSpec((B, KA), lambda j: (0, 0)),
            pl.BlockSpec((D, BN), lambda j: (0, j)),
            pl.BlockSpec((1, BN), lambda j: (0, j)),
        ],
        out_specs=pl.BlockSpec((BN, B), lambda j: (j, 0)),
        out_shape=jax.ShapeDtypeStruct((V, B), jnp.float32),
    )


def kernel(batch, embed_table, W, b):
    B, L = batch.shape
    V, D = embed_table.shape
    tbl_rows = _make_rowize(D, V, 32768)(embed_table.T)
    idx_flat = batch.astype(jnp.int32).reshape(B * L)
    pool, NW = _make_pool(V, D, B, L)
    xa = pool(idx_flat, tbl_rows)
    out_t = _make_matmul(B, D, V, 6144)(xa, W.T, b.reshape(1, V))
    return out_t.T


# speedup vs baseline: 1.0036x; 1.0036x over previous
import functools

import jax
import jax.numpy as jnp
from jax import lax
from jax.experimental import pallas as pl
from jax.experimental.pallas import tpu as pltpu
from jax.experimental.pallas import tpu_sc as plsc

CHUNK = 128
DP = 128
KA = 80


def _make_rowize(D, V, BN):
    def tr(t_ref, o_ref):
        o_ref[:, : D] = jnp.swapaxes(t_ref[...], 0, 1)

    return pl.pallas_call(
        tr,
        grid=(pl.cdiv(V, BN),),
        in_specs=[pl.BlockSpec((D, BN), lambda j: (0, j))],
        out_specs=pl.BlockSpec((BN, DP), lambda j: (j, 0)),
        out_shape=jax.ShapeDtypeStruct((V, DP), jnp.float32),
    )


@functools.lru_cache(maxsize=None)
def _make_pool(V, D, B, L):
    info = plsc.get_sparse_core_info()
    NC, NS, LN = info.num_cores, info.num_subcores, info.num_lanes
    NW = NC * NS
    assert B % NW == 0 and D % LN == 0 and KA % LN == 0
    b_per_w = B // NW
    n_idx = b_per_w * L
    assert n_idx % CHUNK == 0
    n_ch = n_idx // CHUNK
    mesh = plsc.VectorSubcoreMesh(core_axis_name="c", subcore_axis_name="s")

    @functools.partial(
        pl.kernel,
        mesh=mesh,
        out_type=jax.ShapeDtypeStruct((B, KA), jnp.float32),
        scratch_types=[
            pltpu.VMEM((n_idx,), jnp.int32),
            pltpu.VMEM((n_idx, DP), jnp.float32),
            pltpu.VMEM((b_per_w, KA), jnp.float32),
            pltpu.SemaphoreType.DMA,
        ],
    )
    def pool(idx_hbm, tbl_hbm, out_hbm, idx_v, rows_v, acc_v, sem):
        wid = lax.axis_index("s") * NC + lax.axis_index("c")
        pltpu.sync_copy(idx_hbm.at[pl.ds(wid * n_idx, n_idx)], idx_v)
        copies = []
        for j in range(n_ch):
            copies.append(
                pltpu.async_copy(
                    tbl_hbm.at[idx_v.at[pl.ds(j * CHUNK, CHUNK)]],
                    rows_v.at[pl.ds(j * CHUNK, CHUNK)],
                    sem,
                )
            )
        onez = jnp.where(
            lax.iota(jnp.int32, LN) == 0, 1.0, 0.0
        ).astype(jnp.float32)

        def body(k, carry):
            r0 = k * L
            for d in range(D // LN):
                a = rows_v[r0, pl.ds(d * LN, LN)]
                for l in range(1, L):
                    a = a + rows_v[r0 + l, pl.ds(d * LN, LN)]
                acc_v[k, pl.ds(d * LN, LN)] = a * (1.0 / L)
            for d in range(D // LN, KA // LN):
                acc_v[k, pl.ds(d * LN, LN)] = onez if d == D // LN else (
                    jnp.zeros((LN,), jnp.float32)
                )
            return carry

        k_done = 0
        for j in range(n_ch):
            copies[j].wait()
            k_ready = min((CHUNK * (j + 1)) // L, b_per_w)
            if k_ready > k_done:
                lax.fori_loop(k_done, k_ready, body, 0)
                k_done = k_ready
        pltpu.sync_copy(acc_v, out_hbm.at[pl.ds(wid * b_per_w, b_per_w)])

    return pool, NW


def _make_matmul(B, D, V, BN):
    def mm(x_ref, w_ref, b_ref, o_ref):
        lhs = jnp.concatenate(
            [w_ref[...], b_ref[...], jnp.zeros((KA - D - 1, BN), jnp.float32)],
            axis=0,
        )
        o_ref[...] = lax.dot_general(
            lhs,
            x_ref[...],
            (((0,), (1,)), ((), ())),
            preferred_element_type=jnp.float32,
        )

    return pl.pallas_call(
        mm,
        grid=(pl.cdiv(V, BN),),
        in_specs=[
            pl.BlockSpec((B, KA), lambda j: (0, 0)),
            pl.BlockSpec((D, BN), lambda j: (0, j)),
            pl.BlockSpec((1, BN), lambda j: (0, j)),
        ],
        out_specs=pl.BlockSpec((BN, B), lambda j: (j, 0)),
        out_shape=jax.ShapeDtypeStruct((V, B), jnp.float32),
    )


def kernel(batch, embed_table, W, b):
    B, L = batch.shape
    V, D = embed_table.shape
    tbl_rows = _make_rowize(D, V, 32768)(embed_table.T)
    idx_flat = batch.astype(jnp.int32).reshape(B * L)
    pool, NW = _make_pool(V, D, B, L)
    xa = pool(idx_flat, tbl_rows)
    out_t = _make_matmul(B, D, V, 5120)(xa, W.T, b.reshape(1, V))
    return out_t.T
